# Initial kernel scaffold; baseline (speedup 1.0000x reference)
import jax
import jax.numpy as jnp
from jax.experimental import pallas as pl


def kernel(x_init, emb, W, source_p):
    raise NotImplementedError("write your pallas kernel here")


# speedup vs baseline: 42.8043x; 42.8043x over previous
import jax
import jax.numpy as jnp
from jax.experimental import pallas as pl


def _body(x_ref, emb_ref, w_ref, out_ref):
    b = x_ref.shape[0]
    v, d = emb_ref.shape
    x = x_ref[...]
    cols = jax.lax.broadcasted_iota(jnp.int32, (b, v), 1)
    onehot = (cols == x).astype(jnp.float32)
    h = jnp.dot(onehot, emb_ref[...], preferred_element_type=jnp.float32)
    h = h * jnp.float32(1.5)
    logits = jnp.dot(h, w_ref[...], preferred_element_type=jnp.float32)
    m = jnp.max(logits, axis=1, keepdims=True)
    e = jnp.exp(logits - m)
    out_ref[...] = e / jnp.sum(e, axis=1, keepdims=True)


def kernel(x_init, emb, W, source_p):
    del source_p
    b = x_init.shape[0]
    v = emb.shape[0]
    x2d = x_init.reshape(b, 1).astype(jnp.int32)
    return pl.pallas_call(
        _body,
        out_shape=jax.ShapeDtypeStruct((b, v), jnp.float32),
    )(x2d, emb, W)
